# baseline (speedup 1.0000x reference)
import jax
import jax.numpy as jnp
from jax import lax
from jax.experimental import pallas as pl
from jax.experimental.pallas import tpu as pltpu
from jax.experimental.pallas import tpu_sc as plsc

VOCAB = 1000000
BATCH = 16384
HIST = 50
CLASSES = 5
SCALE = 1.0 / 16384

NUM_CORES = 2
NUM_SUBCORES = 16
LANES = 16
NW = NUM_CORES * NUM_SUBCORES
B_PER_W = BATCH // NW
CHUNK = 128
N_CHUNKS = B_PER_W // CHUNK
IDX_PER_CHUNK = CHUNK * HIST
IDX_ROWS = IDX_PER_CHUNK // 128
GROUPS = CHUNK // LANES


def _sc_body(idx_hbm, zeros_hbm, table_hbm, sums_hbm,
             idx_v, rows_v, dst_idx_v, acc_sh, gsem, asem):
    core = lax.axis_index("c")
    sub = lax.axis_index("s")
    wid = sub * NUM_CORES + core
    lane = lax.iota(jnp.int32, LANES)

    for j in range(IDX_ROWS):
        for t in range(128 // LANES):
            r = j * 128 + t * LANES
            q0, rem = divmod(r, HIST)
            step = jnp.where(lane >= (HIST - rem), 1, 0)
            dst_idx_v[j, pl.ds(t * LANES, LANES)] = sub * CHUNK + q0 + step

    def chunk_body(k, _):
        cid = wid * N_CHUNKS + k
        my_acc = acc_sh.at[pl.ds(sub * CHUNK, CHUNK)]

        pltpu.sync_copy(idx_hbm.at[cid], idx_v)
        pltpu.sync_copy(zeros_hbm, my_acc)

        for j in range(IDX_ROWS):
            pltpu.async_copy(
                table_hbm.at[idx_v.at[j]],
                rows_v.at[pl.ds(j * 128, 128)],
                gsem,
            )
        for j in range(IDX_ROWS):
            pltpu.make_async_copy(
                table_hbm.at[idx_v.at[j]],
                rows_v.at[pl.ds(j * 128, 128)],
                gsem,
            ).wait()

        for j in range(IDX_ROWS):
            pltpu.async_copy(
                rows_v.at[pl.ds(j * 128, 128)],
                acc_sh.at[dst_idx_v.at[j]],
                asem,
                add=True,
            )
        for j in range(IDX_ROWS):
            pltpu.make_async_copy(
                rows_v.at[pl.ds(j * 128, 128)],
                acc_sh.at[dst_idx_v.at[j]],
                asem,
            ).wait()

        pltpu.sync_copy(my_acc, sums_hbm.at[cid])
        return ()

    lax.fori_loop(0, N_CHUNKS, chunk_body, ())


@jax.jit
def _embed_sums(idx3d, zeros, table):
    mesh = plsc.VectorSubcoreMesh(
        core_axis_name="c", subcore_axis_name="s",
        num_cores=NUM_CORES, num_subcores=NUM_SUBCORES)
    return pl.kernel(
        _sc_body,
        out_type=jax.ShapeDtypeStruct(
            (NW * N_CHUNKS, CHUNK, CLASSES), jnp.float32),
        mesh=mesh,
        compiler_params=pltpu.CompilerParams(use_tc_tiling_on_sc=False),
        scratch_types=[
            pltpu.VMEM((IDX_ROWS, 128), jnp.int32),
            pltpu.VMEM((IDX_PER_CHUNK, CLASSES), jnp.float32),
            pltpu.VMEM((IDX_ROWS, 128), jnp.int32),
            pltpu.VMEM_SHARED((NUM_SUBCORES * CHUNK, CLASSES), jnp.float32),
            pltpu.SemaphoreType.DMA,
            pltpu.SemaphoreType.DMA,
        ],
    )(idx3d, zeros, table)


def _softmax_body(s_ref, o_ref):
    s = s_ref[...] * SCALE
    m = jnp.max(s, axis=-1, keepdims=True)
    e = jnp.exp(s - m)
    o_ref[...] = e / jnp.sum(e, axis=-1, keepdims=True)


@jax.jit
def _softmax(sums):
    return pl.pallas_call(
        _softmax_body,
        out_shape=jax.ShapeDtypeStruct((BATCH, CLASSES), jnp.float32),
        grid=(8,),
        in_specs=[pl.BlockSpec((BATCH // 8, CLASSES), lambda i: (i, 0))],
        out_specs=pl.BlockSpec((BATCH // 8, CLASSES), lambda i: (i, 0)),
    )(sums)


def kernel(indices, table):
    idx3d = indices.reshape(
        NW * N_CHUNKS, IDX_ROWS, 128).astype(jnp.int32)
    zeros = jnp.zeros((CHUNK, CLASSES), jnp.float32)
    sums = _embed_sums(idx3d, zeros, table).reshape(BATCH, CLASSES)
    return _softmax(sums)


# speedup vs baseline: 1.0077x; 1.0077x over previous
import jax
import jax.numpy as jnp
from jax import lax
from jax.experimental import pallas as pl
from jax.experimental.pallas import tpu as pltpu
from jax.experimental.pallas import tpu_sc as plsc

VOCAB = 1000000
BATCH = 16384
HIST = 50
CLASSES = 5
SCALE = 1.0 / 16384

NUM_CORES = 2
NUM_SUBCORES = 16
LANES = 16
NW = NUM_CORES * NUM_SUBCORES
B_PER_W = BATCH // NW
CHUNK = 128
N_CHUNKS = B_PER_W // CHUNK
IDX_PER_CHUNK = CHUNK * HIST
IDX_ROWS = IDX_PER_CHUNK // 128
GROUPS = CHUNK // LANES
PADC = 8


def _sc_body(idx_hbm, zeros_hbm, table_hbm, sums_hbm,
             idx_v, rows_v, dst_idx_v, acc_sh, gsem, asem):
    core = lax.axis_index("c")
    sub = lax.axis_index("s")
    wid = sub * NUM_CORES + core
    lane = lax.iota(jnp.int32, LANES)

    for j in range(IDX_ROWS):
        for t in range(128 // LANES):
            r = j * 128 + t * LANES
            q0, rem = divmod(r, HIST)
            step = jnp.where(lane >= (HIST - rem), 1, 0)
            dst_idx_v[j, pl.ds(t * LANES, LANES)] = sub * CHUNK + q0 + step

    def chunk_body(k, _):
        cid = wid * N_CHUNKS + k
        my_acc = acc_sh.at[pl.ds(sub * CHUNK, CHUNK)]

        pltpu.sync_copy(idx_hbm.at[cid], idx_v)
        pltpu.sync_copy(zeros_hbm, my_acc)

        for j in range(IDX_ROWS):
            pltpu.async_copy(
                table_hbm.at[idx_v.at[j]],
                rows_v.at[pl.ds(j * 128, 128)],
                gsem,
            )
        for j in range(IDX_ROWS):
            pltpu.make_async_copy(
                table_hbm.at[idx_v.at[j]],
                rows_v.at[pl.ds(j * 128, 128)],
                gsem,
            ).wait()

        for j in range(IDX_ROWS):
            pltpu.async_copy(
                rows_v.at[pl.ds(j * 128, 128)],
                acc_sh.at[dst_idx_v.at[j]],
                asem,
                add=True,
            )
        for j in range(IDX_ROWS):
            pltpu.make_async_copy(
                rows_v.at[pl.ds(j * 128, 128)],
                acc_sh.at[dst_idx_v.at[j]],
                asem,
            ).wait()

        pltpu.sync_copy(my_acc, sums_hbm.at[cid])
        return ()

    lax.fori_loop(0, N_CHUNKS, chunk_body, ())


@jax.jit
def _embed_sums(idx3d, zeros, table):
    mesh = plsc.VectorSubcoreMesh(
        core_axis_name="c", subcore_axis_name="s",
        num_cores=NUM_CORES, num_subcores=NUM_SUBCORES)
    return pl.kernel(
        _sc_body,
        out_type=jax.ShapeDtypeStruct(
            (NW * N_CHUNKS, CHUNK, PADC), jnp.float32),
        mesh=mesh,
        compiler_params=pltpu.CompilerParams(use_tc_tiling_on_sc=False),
        scratch_types=[
            pltpu.VMEM((IDX_ROWS, 128), jnp.int32),
            pltpu.VMEM((IDX_PER_CHUNK, PADC), jnp.float32),
            pltpu.VMEM((IDX_ROWS, 128), jnp.int32),
            pltpu.VMEM_SHARED((NUM_SUBCORES * CHUNK, PADC), jnp.float32),
            pltpu.SemaphoreType.DMA,
            pltpu.SemaphoreType.DMA,
        ],
    )(idx3d, zeros, table)


def _softmax_body(s_ref, o_ref):
    s = s_ref[:, :CLASSES] * SCALE
    m = jnp.max(s, axis=-1, keepdims=True)
    e = jnp.exp(s - m)
    o_ref[...] = e / jnp.sum(e, axis=-1, keepdims=True)


@jax.jit
def _softmax(sums):
    return pl.pallas_call(
        _softmax_body,
        out_shape=jax.ShapeDtypeStruct((BATCH, CLASSES), jnp.float32),
        grid=(8,),
        in_specs=[pl.BlockSpec((BATCH // 8, PADC), lambda i: (i, 0))],
        out_specs=pl.BlockSpec((BATCH // 8, CLASSES), lambda i: (i, 0)),
    )(sums)


def kernel(indices, table):
    idx3d = indices.reshape(
        NW * N_CHUNKS, IDX_ROWS, 128).astype(jnp.int32)
    zeros = jnp.zeros((CHUNK, PADC), jnp.float32)
    table8 = jnp.pad(table, ((0, 0), (0, PADC - CLASSES)))
    sums = _embed_sums(idx3d, zeros, table8).reshape(BATCH, PADC)
    return _softmax(sums)


# speedup vs baseline: 1.0988x; 1.0905x over previous
import jax
import jax.numpy as jnp
from jax import lax
from jax.experimental import pallas as pl
from jax.experimental.pallas import tpu as pltpu
from jax.experimental.pallas import tpu_sc as plsc

VOCAB = 1000000
BATCH = 16384
HIST = 50
CLASSES = 5
SCALE = 1.0 / 16384

NUM_CORES = 2
NUM_SUBCORES = 16
LANES = 16
NW = NUM_CORES * NUM_SUBCORES
B_PER_W = BATCH // NW
CHUNK = 16
N_CHUNKS = B_PER_W // CHUNK
IDX_PER_CHUNK = CHUNK * HIST
SW = 16
IDX_ROWS = IDX_PER_CHUNK // SW
PADC = 8
WIDE = 128


def _sc_body(idx_hbm, zeros_hbm, table_hbm, sums_hbm,
             idx_v, rows_v, comp_v, dst_idx_v, acc_sh, bounce_sh,
             gsem, asem):
    core = lax.axis_index("c")
    sub = lax.axis_index("s")
    wid = sub * NUM_CORES + core
    lane = lax.iota(jnp.int32, LANES)

    for j in range(IDX_ROWS):
        q0, rem = divmod(j * SW, HIST)
        step = jnp.where(lane >= (HIST - rem), 1, 0)
        dst_idx_v[j, pl.ds(0, LANES)] = sub * CHUNK + q0 + step

    def chunk_body(k, _):
        cid = wid * N_CHUNKS + k
        my_acc = acc_sh.at[pl.ds(sub * CHUNK, CHUNK)]

        pltpu.sync_copy(idx_hbm.at[cid], idx_v)
        pltpu.sync_copy(zeros_hbm, my_acc)

        for j in range(IDX_ROWS):
            pltpu.async_copy(
                table_hbm.at[idx_v.at[j]],
                rows_v.at[pl.ds(j * SW, SW)],
                gsem,
            )
        for j in range(IDX_ROWS):
            pltpu.make_async_copy(
                table_hbm.at[idx_v.at[j]],
                rows_v.at[pl.ds(j * SW, SW)],
                gsem,
            ).wait()

        my_bounce = bounce_sh.at[pl.ds(sub * IDX_PER_CHUNK, IDX_PER_CHUNK)]
        pltpu.sync_copy(rows_v.at[:, pl.ds(0, PADC)], my_bounce)
        pltpu.sync_copy(my_bounce, comp_v)

        for j in range(IDX_ROWS):
            pltpu.async_copy(
                comp_v.at[pl.ds(j * SW, SW)],
                acc_sh.at[dst_idx_v.at[j]],
                asem,
                add=True,
            )
        for j in range(IDX_ROWS):
            pltpu.make_async_copy(
                comp_v.at[pl.ds(j * SW, SW)],
                acc_sh.at[dst_idx_v.at[j]],
                asem,
            ).wait()

        pltpu.sync_copy(my_acc, sums_hbm.at[cid])
        return ()

    lax.fori_loop(0, N_CHUNKS, chunk_body, ())


@jax.jit
def _embed_sums(idx3d, zeros, table):
    mesh = plsc.VectorSubcoreMesh(
        core_axis_name="c", subcore_axis_name="s",
        num_cores=NUM_CORES, num_subcores=NUM_SUBCORES)
    return pl.kernel(
        _sc_body,
        out_type=jax.ShapeDtypeStruct(
            (NW * N_CHUNKS, CHUNK, PADC), jnp.float32),
        mesh=mesh,
        compiler_params=pltpu.CompilerParams(use_tc_tiling_on_sc=False),
        scratch_types=[
            pltpu.VMEM((IDX_ROWS, SW), jnp.int32),
            pltpu.VMEM((IDX_PER_CHUNK, WIDE), jnp.float32),
            pltpu.VMEM((IDX_PER_CHUNK, PADC), jnp.float32),
            pltpu.VMEM((IDX_ROWS, SW), jnp.int32),
            pltpu.VMEM_SHARED((NUM_SUBCORES * CHUNK, PADC), jnp.float32),
            pltpu.VMEM_SHARED(
                (NUM_SUBCORES * IDX_PER_CHUNK, PADC), jnp.float32),
            pltpu.SemaphoreType.DMA,
            pltpu.SemaphoreType.DMA,
        ],
    )(idx3d, zeros, table)


def _softmax_body(s_ref, o_ref):
    s = s_ref[:, :CLASSES] * SCALE
    m = jnp.max(s, axis=-1, keepdims=True)
    e = jnp.exp(s - m)
    o_ref[...] = e / jnp.sum(e, axis=-1, keepdims=True)


@jax.jit
def _softmax(sums):
    return pl.pallas_call(
        _softmax_body,
        out_shape=jax.ShapeDtypeStruct((BATCH, CLASSES), jnp.float32),
        grid=(8,),
        in_specs=[pl.BlockSpec((BATCH // 8, PADC), lambda i: (i, 0))],
        out_specs=pl.BlockSpec((BATCH // 8, CLASSES), lambda i: (i, 0)),
    )(sums)


def kernel(indices, table):
    idx3d = indices.reshape(
        NW * N_CHUNKS, IDX_ROWS, SW).astype(jnp.int32)
    zeros = jnp.zeros((CHUNK, PADC), jnp.float32)
    tablew = jnp.pad(table, ((0, 0), (0, WIDE - CLASSES)))
    sums = _embed_sums(idx3d, zeros, tablew).reshape(BATCH, PADC)
    return _softmax(sums)
